# baseline (speedup 1.0000x reference)
import jax
import jax.numpy as jnp
from jax import lax
from jax.experimental import pallas as pl
from jax.experimental.pallas import tpu as pltpu

_B_BLK = 16


def _fused(xa_ref, xb_ref, m_ref, w1a_ref, w1b_ref, b1_ref, w2_ref, b2_ref,
           w3_ref, b3_ref, o_ref):
    n_rows = xa_ref.shape[0] * xa_ref.shape[1]
    N = xa_ref.shape[1]
    xa = xa_ref[...].reshape(n_rows, xa_ref.shape[2])
    xb = xb_ref[...].reshape(n_rows, xb_ref.shape[2])
    h = jnp.dot(xa, w1a_ref[...], preferred_element_type=jnp.float32)
    h = h + jnp.dot(xb, w1b_ref[...], preferred_element_type=jnp.float32)
    h = jnp.maximum(h + b1_ref[...], 0.0)
    h = jnp.maximum(
        jnp.dot(h, w2_ref[...], preferred_element_type=jnp.float32)
        + b2_ref[...], 0.0)
    for j in range(_B_BLK):
        h2j = h[j * N:(j + 1) * N, :]
        t = lax.dot_general(
            w3_ref[...], h2j,
            dimension_numbers=(((1,), (1,)), ((), ())),
            preferred_element_type=jnp.float32)
        t = t + b3_ref[0, 0]
        m = m_ref[0, j:j + 1, :]
        t = t * m - 999.0 * (1.0 - m)
        t = t - jnp.max(t, axis=1, keepdims=True)
        e = jnp.exp(t)
        o_ref[0, j:j + 1, :] = e / jnp.sum(e, axis=1, keepdims=True)


def kernel(inputs, mask, W1, b1, W2, b2, W3, b3):
    B, N, D = inputs.shape
    H1 = W1.shape[1]
    H2 = W2.shape[1]
    Dh = D // 2
    mf = mask.astype(jnp.float32).reshape(B // _B_BLK, _B_BLK, N)
    b1r = b1.reshape(1, H1)
    b2r = b2.reshape(1, H2)
    b3r = b3.reshape(1, 1)
    w3r = W3.reshape(1, H2)

    out = pl.pallas_call(
        _fused,
        grid=(B // _B_BLK,),
        in_specs=[
            pl.BlockSpec((_B_BLK, N, Dh), lambda i: (i, 0, 0)),
            pl.BlockSpec((_B_BLK, N, Dh), lambda i: (i, 0, 1)),
            pl.BlockSpec((1, _B_BLK, N), lambda i: (i, 0, 0)),
            pl.BlockSpec((Dh, H1), lambda i: (0, 0)),
            pl.BlockSpec((Dh, H1), lambda i: (1, 0)),
            pl.BlockSpec((1, H1), lambda i: (0, 0)),
            pl.BlockSpec((H1, H2), lambda i: (0, 0)),
            pl.BlockSpec((1, H2), lambda i: (0, 0)),
            pl.BlockSpec((1, H2), lambda i: (0, 0)),
            pl.BlockSpec((1, 1), lambda i: (0, 0)),
        ],
        out_specs=pl.BlockSpec((1, _B_BLK, N), lambda i: (i, 0, 0)),
        out_shape=jax.ShapeDtypeStruct((B // _B_BLK, _B_BLK, N), jnp.float32),
        compiler_params=pltpu.CompilerParams(
            dimension_semantics=("parallel",),
        ),
    )(inputs, inputs, mf, W1, W1, b1r, W2, b2r, w3r, b3r)
    return out.reshape(B, N, 1)


# speedup vs baseline: 1.3893x; 1.3893x over previous
import jax
import jax.numpy as jnp
from jax import lax
from jax.experimental import pallas as pl
from jax.experimental.pallas import tpu as pltpu

_B_BLK = 16


def _mlp_softmax(x_ref, m_ref, w1_ref, b1_ref, w2_ref, b2_ref, w3_ref, b3_ref,
                 o_ref):
    n_rows = x_ref.shape[0] * x_ref.shape[1]
    N = x_ref.shape[1]
    x = x_ref[...].reshape(n_rows, x_ref.shape[2])
    h = jnp.maximum(
        jnp.dot(x, w1_ref[...], preferred_element_type=jnp.float32)
        + b1_ref[...], 0.0)
    h = jnp.maximum(
        jnp.dot(h, w2_ref[...], preferred_element_type=jnp.float32)
        + b2_ref[...], 0.0)
    for j in range(_B_BLK):
        h2j = h[j * N:(j + 1) * N, :]
        t = lax.dot_general(
            w3_ref[...], h2j,
            dimension_numbers=(((1,), (1,)), ((), ())),
            preferred_element_type=jnp.float32)
        t = t + b3_ref[0, 0]
        m = m_ref[0, j:j + 1, :]
        t = t * m - 999.0 * (1.0 - m)
        t = t - jnp.max(t, axis=1, keepdims=True)
        e = jnp.exp(t)
        o_ref[0, j:j + 1, :] = e / jnp.sum(e, axis=1, keepdims=True)


def _fused(xa_ref, xb_ref, ma_ref, mb_ref, w1_ref, b1_ref, w2_ref, b2_ref,
           w3_ref, b3_ref, oa_ref, ob_ref):
    _mlp_softmax(xa_ref, ma_ref, w1_ref, b1_ref, w2_ref, b2_ref, w3_ref,
                 b3_ref, oa_ref)
    _mlp_softmax(xb_ref, mb_ref, w1_ref, b1_ref, w2_ref, b2_ref, w3_ref,
                 b3_ref, ob_ref)


def kernel(inputs, mask, W1, b1, W2, b2, W3, b3):
    B, N, D = inputs.shape
    H1 = W1.shape[1]
    H2 = W2.shape[1]
    G = B // (2 * _B_BLK)
    mf = mask.astype(jnp.float32).reshape(G, 2, _B_BLK, N)
    ma = mf[:, 0]
    mb = mf[:, 1]
    b1r = b1.reshape(1, H1)
    b2r = b2.reshape(1, H2)
    b3r = b3.reshape(1, 1)
    w3r = W3.reshape(1, H2)

    io_shape = jax.ShapeDtypeStruct((G, _B_BLK, N), jnp.float32)
    oa, ob = pl.pallas_call(
        _fused,
        grid=(G,),
        in_specs=[
            pl.BlockSpec((_B_BLK, N, D), lambda i: (2 * i, 0, 0)),
            pl.BlockSpec((_B_BLK, N, D), lambda i: (2 * i + 1, 0, 0)),
            pl.BlockSpec((1, _B_BLK, N), lambda i: (i, 0, 0)),
            pl.BlockSpec((1, _B_BLK, N), lambda i: (i, 0, 0)),
            pl.BlockSpec((D, H1), lambda i: (0, 0)),
            pl.BlockSpec((1, H1), lambda i: (0, 0)),
            pl.BlockSpec((H1, H2), lambda i: (0, 0)),
            pl.BlockSpec((1, H2), lambda i: (0, 0)),
            pl.BlockSpec((1, H2), lambda i: (0, 0)),
            pl.BlockSpec((1, 1), lambda i: (0, 0)),
        ],
        out_specs=[
            pl.BlockSpec((1, _B_BLK, N), lambda i: (i, 0, 0)),
            pl.BlockSpec((1, _B_BLK, N), lambda i: (i, 0, 0)),
        ],
        out_shape=[io_shape, io_shape],
        compiler_params=pltpu.CompilerParams(
            dimension_semantics=("parallel",),
        ),
    )(inputs, inputs, ma, mb, W1, b1r, W2, b2r, w3r, b3r)
    out = jnp.stack([oa, ob], axis=1)
    return out.reshape(B, N, 1)


# speedup vs baseline: 1.4909x; 1.0731x over previous
import jax
import jax.numpy as jnp
from jax import lax
from jax.experimental import pallas as pl
from jax.experimental.pallas import tpu as pltpu

_B_BLK = 32


def _fused(x_ref, m_ref, w1_ref, b1_ref, w2_ref, b2_ref, w3_ref, b3_ref, o_ref):
    n_rows = x_ref.shape[0] * x_ref.shape[1]
    N = x_ref.shape[1]
    x = x_ref[...].reshape(n_rows, x_ref.shape[2])
    h = jnp.maximum(
        jnp.dot(x, w1_ref[...], preferred_element_type=jnp.float32)
        + b1_ref[...], 0.0)
    h = jnp.maximum(
        jnp.dot(h, w2_ref[...], preferred_element_type=jnp.float32)
        + b2_ref[...], 0.0)
    for j in range(_B_BLK):
        h2j = h[j * N:(j + 1) * N, :]
        t = lax.dot_general(
            w3_ref[...], h2j,
            dimension_numbers=(((1,), (1,)), ((), ())),
            preferred_element_type=jnp.float32)
        t = t + b3_ref[0, 0]
        m = m_ref[0, j:j + 1, :]
        t = t * m - 999.0 * (1.0 - m)
        t = t - jnp.max(t, axis=1, keepdims=True)
        e = jnp.exp(t)
        o_ref[0, j:j + 1, :] = e / jnp.sum(e, axis=1, keepdims=True)


def kernel(inputs, mask, W1, b1, W2, b2, W3, b3):
    B, N, D = inputs.shape
    H1 = W1.shape[1]
    H2 = W2.shape[1]
    mf = mask.astype(jnp.float32).reshape(B // _B_BLK, _B_BLK, N)
    b1r = b1.reshape(1, H1)
    b2r = b2.reshape(1, H2)
    b3r = b3.reshape(1, 1)
    w3r = W3.reshape(1, H2)

    out = pl.pallas_call(
        _fused,
        grid=(B // _B_BLK,),
        in_specs=[
            pl.BlockSpec((_B_BLK, N, D), lambda i: (i, 0, 0)),
            pl.BlockSpec((1, _B_BLK, N), lambda i: (i, 0, 0)),
            pl.BlockSpec((D, H1), lambda i: (0, 0)),
            pl.BlockSpec((1, H1), lambda i: (0, 0)),
            pl.BlockSpec((H1, H2), lambda i: (0, 0)),
            pl.BlockSpec((1, H2), lambda i: (0, 0)),
            pl.BlockSpec((1, H2), lambda i: (0, 0)),
            pl.BlockSpec((1, 1), lambda i: (0, 0)),
        ],
        out_specs=pl.BlockSpec((1, _B_BLK, N), lambda i: (i, 0, 0)),
        out_shape=jax.ShapeDtypeStruct((B // _B_BLK, _B_BLK, N), jnp.float32),
        compiler_params=pltpu.CompilerParams(
            dimension_semantics=("arbitrary",),
        ),
    )(inputs, mf, W1, b1r, W2, b2r, w3r, b3r)
    return out.reshape(B, N, 1)


# speedup vs baseline: 1.5697x; 1.0529x over previous
import jax
import jax.numpy as jnp
from jax import lax
from jax.experimental import pallas as pl
from jax.experimental.pallas import tpu as pltpu

_B_BLK = 32


def _fused(x_ref, m_ref, w1_ref, b1_ref, w2_ref, b2_ref, w3_ref, b3_ref, o_ref):
    n_rows = x_ref.shape[0] * x_ref.shape[1]
    N = x_ref.shape[1]
    x = x_ref[...].reshape(n_rows, x_ref.shape[2])
    h = jnp.maximum(
        jnp.dot(x, w1_ref[...], preferred_element_type=jnp.float32)
        + b1_ref[...], 0.0)
    h = jnp.maximum(
        jnp.dot(h, w2_ref[...], preferred_element_type=jnp.float32)
        + b2_ref[...], 0.0)
    for j in range(_B_BLK):
        h2j = h[j * N:(j + 1) * N, :]
        t = lax.dot_general(
            w3_ref[...], h2j,
            dimension_numbers=(((1,), (1,)), ((), ())),
            preferred_element_type=jnp.float32)
        t = t + b3_ref[0, 0]
        m = m_ref[j:j + 1, :].astype(jnp.float32)
        t = t * m - 999.0 * (1.0 - m)
        t = t - jnp.max(t, axis=1, keepdims=True)
        e = jnp.exp(t)
        o_ref[j:j + 1, :] = e / jnp.sum(e, axis=1, keepdims=True)


def kernel(inputs, mask, W1, b1, W2, b2, W3, b3):
    B, N, D = inputs.shape
    H1 = W1.shape[1]
    H2 = W2.shape[1]
    b1r = b1.reshape(1, H1)
    b2r = b2.reshape(1, H2)
    b3r = b3.reshape(1, 1)
    w3r = W3.reshape(1, H2)

    out = pl.pallas_call(
        _fused,
        grid=(B // _B_BLK,),
        in_specs=[
            pl.BlockSpec((_B_BLK, N, D), lambda i: (i, 0, 0)),
            pl.BlockSpec((_B_BLK, N), lambda i: (i, 0)),
            pl.BlockSpec((D, H1), lambda i: (0, 0)),
            pl.BlockSpec((1, H1), lambda i: (0, 0)),
            pl.BlockSpec((H1, H2), lambda i: (0, 0)),
            pl.BlockSpec((1, H2), lambda i: (0, 0)),
            pl.BlockSpec((1, H2), lambda i: (0, 0)),
            pl.BlockSpec((1, 1), lambda i: (0, 0)),
        ],
        out_specs=pl.BlockSpec((_B_BLK, N), lambda i: (i, 0)),
        out_shape=jax.ShapeDtypeStruct((B, N), jnp.float32),
        compiler_params=pltpu.CompilerParams(
            dimension_semantics=("parallel",),
        ),
    )(inputs, mask, W1, b1r, W2, b2r, w3r, b3r)
    return out.reshape(B, N, 1)
